# initial kernel scaffold (unmeasured)
import jax
import jax.numpy as jnp
from jax import lax
from jax.experimental import pallas as pl
from jax.experimental.pallas import tpu as pltpu

N_DEV = 8
B = 2
SQ = 512
SKV = 512
D_MODEL = 768
H_PER = 8
DH = 64
D_CHUNK = H_PER * DH
SCALE = 0.125
NEG = -1e9


def kernel(x, Wq, K_ext, V_ext, Wo):
    K_r = K_ext.reshape(B, SKV, N_DEV, H_PER, DH).transpose(2, 0, 3, 1, 4)
    V_r = V_ext.reshape(B, SKV, N_DEV, H_PER, DH).transpose(2, 0, 3, 1, 4)

    def body(x_ref, wq_ref, k_ref, v_ref, wo_ref, out_ref,
             comm_wq, comm_wo, send_wq, recv_wq, send_wo, recv_wo):
        my = lax.axis_index("i")
        left = (my - 1) % N_DEV
        right = (my + 1) % N_DEV

        barrier = pltpu.get_barrier_semaphore()
        for nbr in (left, right):
            pl.semaphore_signal(barrier, inc=1, device_id=(nbr,),
                                device_id_type=pl.DeviceIdType.MESH)
        pl.semaphore_wait(barrier, 2)

        def rdma(src, c):
            return pltpu.make_async_remote_copy(
                src_ref=src,
                dst_ref=comm_wq.at[c] if src is not wo_src[0] else None,
                send_sem=send_wq.at[c], recv_sem=recv_wq.at[c],
                device_id=(right,), device_id_type=pl.DeviceIdType.MESH)

        def rdma_wq(src, c):
            return pltpu.make_async_remote_copy(
                src_ref=src, dst_ref=comm_wq.at[c],
                send_sem=send_wq.at[c], recv_sem=recv_wq.at[c],
                device_id=(right,), device_id_type=pl.DeviceIdType.MESH)

        def rdma_wo(src, c):
            return pltpu.make_async_remote_copy(
                src_ref=src, dst_ref=comm_wo.at[c],
                send_sem=send_wo.at[c], recv_sem=recv_wo.at[c],
                device_id=(right,), device_id_type=pl.DeviceIdType.MESH)

        rdma_wq(wq_ref, my).start()
        rdma_wo(wo_ref, my).start()

        rows = lax.broadcasted_iota(jnp.int32, (SQ, SKV), 0) // DH
        cols = lax.broadcasted_iota(jnp.int32, (SQ, SKV), 1) // DH
        mask = (rows % 4) == (cols % 4)
        mask = jnp.logical_or(mask, jnp.logical_and(rows == cols, my == 0))
        bias = jnp.where(mask, 0.0, NEG).astype(jnp.float32)

        for h in range(N_DEV):
            r = (my - h) % N_DEV
            if h > 0:
                d_wq = rdma_wq(comm_wq.at[r], r)
                d_wo = rdma_wo(comm_wo.at[r], r)
                d_wq.wait_recv()
                d_wo.wait_recv()
                if h < N_DEV - 1:
                    d_wq.start()
                    d_wo.start()
                wq_c = comm_wq[r]
                wo_c = comm_wo[r]
            else:
                wq_c = wq_ref[...]
                wo_c = wo_ref[...]

            for b in range(B):
                q_b = lax.dot_general(
                    x_ref[b], wq_c, (((1,), (0,)), ((), ())),
                    preferred_element_type=jnp.float32)
                ctx_parts = []
                for hl in range(H_PER):
                    qh = q_b[:, hl * DH:(hl + 1) * DH]
                    kh = k_ref[r, b, hl]
                    vh = v_ref[r, b, hl]
                    s = lax.dot_general(
                        qh, kh, (((1,), (1,)), ((), ())),
                        preferred_element_type=jnp.float32)
                    s = s * SCALE + bias
                    m = jnp.max(s, axis=-1, keepdims=True)
                    e = jnp.exp(s - m)
                    w = e / jnp.sum(e, axis=-1, keepdims=True)
                    ctx_parts.append(lax.dot_general(
                        w, vh, (((1,), (0,)), ((), ())),
                        preferred_element_type=jnp.float32))
                ctx_b = jnp.concatenate(ctx_parts, axis=1)
                partial = lax.dot_general(
                    ctx_b, wo_c, (((1,), (0,)), ((), ())),
                    preferred_element_type=jnp.float32)
                if h == 0:
                    out_ref[b, :, :] = partial
                else:
                    out_ref[b, :, :] = out_ref[b, :, :] + partial

        for h in range(N_DEV - 1):
            c = (my - h) % N_DEV
            rdma_wq(comm_wq.at[c], c).wait_send()
            rdma_wo(comm_wo.at[c], c).wait_send()

    return pl.pallas_call(
        body,
        out_shape=jax.ShapeDtypeStruct((B, SQ, D_MODEL), jnp.float32),
        in_specs=[pl.BlockSpec(memory_space=pltpu.VMEM)] * 5,
        out_specs=pl.BlockSpec(memory_space=pltpu.VMEM),
        scratch_shapes=[
            pltpu.VMEM((N_DEV, D_MODEL, D_CHUNK), jnp.float32),
            pltpu.VMEM((N_DEV, D_CHUNK, D_MODEL), jnp.float32),
            pltpu.SemaphoreType.DMA((N_DEV,)),
            pltpu.SemaphoreType.DMA((N_DEV,)),
            pltpu.SemaphoreType.DMA((N_DEV,)),
            pltpu.SemaphoreType.DMA((N_DEV,)),
        ],
        compiler_params=pltpu.CompilerParams(collective_id=0),
    )(x, Wq, K_r, V_r, Wo)


# baseline (device time: 302201 ns/iter reference)
import jax
import jax.numpy as jnp
from jax import lax
from jax.experimental import pallas as pl
from jax.experimental.pallas import tpu as pltpu

N_DEV = 8
B = 2
SQ = 512
SKV = 512
D_MODEL = 768
H_PER = 8
DH = 64
D_CHUNK = H_PER * DH
SCALE = 0.125
NEG = -1e9


def kernel(x, Wq, K_ext, V_ext, Wo):
    K_r = K_ext.reshape(B, SKV, N_DEV, D_CHUNK).transpose(2, 0, 1, 3)
    V_r = V_ext.reshape(B, SKV, N_DEV, D_CHUNK).transpose(2, 0, 1, 3)

    def body(x_ref, wq_ref, k_hbm, v_hbm, wo_ref, out_ref,
             comm_wq, comm_wo, k_buf, v_buf,
             send_wq, recv_wq, send_wo, recv_wo, k_sem, v_sem):
        my = lax.axis_index("i")
        left = (my - 1) % N_DEV
        right = (my + 1) % N_DEV

        def kv_fetch(c, slot):
            return (
                pltpu.make_async_copy(k_hbm.at[c], k_buf.at[slot], k_sem.at[slot]),
                pltpu.make_async_copy(v_hbm.at[c], v_buf.at[slot], v_sem.at[slot]),
            )

        for d in kv_fetch(my, 0):
            d.start()

        barrier = pltpu.get_barrier_semaphore()
        for nbr in (left, right):
            pl.semaphore_signal(barrier, inc=1, device_id=(nbr,),
                                device_id_type=pl.DeviceIdType.MESH)
        pl.semaphore_wait(barrier, 2)

        def rdma_wq(src, c):
            return pltpu.make_async_remote_copy(
                src_ref=src, dst_ref=comm_wq.at[c],
                send_sem=send_wq.at[c], recv_sem=recv_wq.at[c],
                device_id=(right,), device_id_type=pl.DeviceIdType.MESH)

        def rdma_wo(src, c):
            return pltpu.make_async_remote_copy(
                src_ref=src, dst_ref=comm_wo.at[c],
                send_sem=send_wo.at[c], recv_sem=recv_wo.at[c],
                device_id=(right,), device_id_type=pl.DeviceIdType.MESH)

        rdma_wq(wq_ref, my).start()
        rdma_wo(wo_ref, my).start()

        rows = lax.broadcasted_iota(jnp.int32, (SQ, SKV), 0) // DH
        cols = lax.broadcasted_iota(jnp.int32, (SQ, SKV), 1) // DH
        mask = (rows % 4) == (cols % 4)
        mask = jnp.logical_or(mask, jnp.logical_and(rows == cols, my == 0))
        bias = jnp.where(mask, 0.0, NEG).astype(jnp.float32)

        for h in range(N_DEV):
            r = (my - h) % N_DEV
            slot = h % 2
            if h > 0:
                d_wq = rdma_wq(comm_wq.at[r], r)
                d_wo = rdma_wo(comm_wo.at[r], r)
                d_wq.wait_recv()
                d_wo.wait_recv()
                if h < N_DEV - 1:
                    d_wq.start()
                    d_wo.start()
                wq_c = comm_wq[r]
                wo_c = comm_wo[r]
            else:
                wq_c = wq_ref[...]
                wo_c = wo_ref[...]

            for d in kv_fetch(r, slot):
                d.wait()
            if h < N_DEV - 1:
                for d in kv_fetch((my - h - 1) % N_DEV, 1 - slot):
                    d.start()

            for b in range(B):
                q_b = lax.dot_general(
                    x_ref[b], wq_c, (((1,), (0,)), ((), ())),
                    preferred_element_type=jnp.float32)
                k_cb = k_buf[slot, b]
                v_cb = v_buf[slot, b]
                ctx_parts = []
                for hl in range(H_PER):
                    qh = q_b[:, hl * DH:(hl + 1) * DH]
                    kh = k_cb[:, hl * DH:(hl + 1) * DH]
                    vh = v_cb[:, hl * DH:(hl + 1) * DH]
                    s = lax.dot_general(
                        qh, kh, (((1,), (1,)), ((), ())),
                        preferred_element_type=jnp.float32)
                    s = s * SCALE + bias
                    m = jnp.max(s, axis=-1, keepdims=True)
                    e = jnp.exp(s - m)
                    w = e / jnp.sum(e, axis=-1, keepdims=True)
                    ctx_parts.append(lax.dot_general(
                        w, vh, (((1,), (0,)), ((), ())),
                        preferred_element_type=jnp.float32))
                ctx_b = jnp.concatenate(ctx_parts, axis=1)
                partial = lax.dot_general(
                    ctx_b, wo_c, (((1,), (0,)), ((), ())),
                    preferred_element_type=jnp.float32)
                if h == 0:
                    out_ref[b, :, :] = partial
                else:
                    out_ref[b, :, :] = out_ref[b, :, :] + partial

        for h in range(N_DEV - 1):
            c = (my - h) % N_DEV
            rdma_wq(comm_wq.at[c], c).wait_send()
            rdma_wo(comm_wo.at[c], c).wait_send()

    return pl.pallas_call(
        body,
        out_shape=jax.ShapeDtypeStruct((B, SQ, D_MODEL), jnp.float32),
        in_specs=[
            pl.BlockSpec(memory_space=pltpu.VMEM),
            pl.BlockSpec(memory_space=pltpu.VMEM),
            pl.BlockSpec(memory_space=pltpu.MemorySpace.HBM),
            pl.BlockSpec(memory_space=pltpu.MemorySpace.HBM),
            pl.BlockSpec(memory_space=pltpu.VMEM),
        ],
        out_specs=pl.BlockSpec(memory_space=pltpu.VMEM),
        scratch_shapes=[
            pltpu.VMEM((N_DEV, D_MODEL, D_CHUNK), jnp.float32),
            pltpu.VMEM((N_DEV, D_CHUNK, D_MODEL), jnp.float32),
            pltpu.VMEM((2, B, SKV, D_CHUNK), jnp.float32),
            pltpu.VMEM((2, B, SKV, D_CHUNK), jnp.float32),
            pltpu.SemaphoreType.DMA((N_DEV,)),
            pltpu.SemaphoreType.DMA((N_DEV,)),
            pltpu.SemaphoreType.DMA((N_DEV,)),
            pltpu.SemaphoreType.DMA((N_DEV,)),
            pltpu.SemaphoreType.DMA((2,)),
            pltpu.SemaphoreType.DMA((2,)),
        ],
        compiler_params=pltpu.CompilerParams(
            collective_id=0,
            vmem_limit_bytes=100 * 1024 * 1024,
        ),
    )(x, Wq, K_r, V_r, Wo)


# device time: 178192 ns/iter; 1.6959x vs baseline; 1.6959x over previous
import jax
import jax.numpy as jnp
from jax import lax
from jax.experimental import pallas as pl
from jax.experimental.pallas import tpu as pltpu

N_DEV = 8
B = 2
SQ = 512
SKV = 512
D_MODEL = 768
H_PER = 8
DH = 64
D_CHUNK = H_PER * DH
SCALE = 0.125
NEG = -1e9
BF = jnp.bfloat16
F32 = jnp.float32


def kernel(x, Wq, K_ext, V_ext, Wo):
    x_b16 = x.astype(BF)
    Wq_b16 = Wq.astype(BF)
    Wo_b16 = Wo.astype(BF)
    K_r = K_ext.reshape(B, SKV, N_DEV, D_CHUNK).transpose(2, 0, 1, 3).astype(BF)
    V_r = V_ext.reshape(B, SKV, N_DEV, D_CHUNK).transpose(2, 0, 1, 3).astype(BF)

    def body(x_ref, wq_ref, k_hbm, v_hbm, wo_ref, out_ref,
             comm_wq, comm_wo, k_buf, v_buf,
             send_wq, recv_wq, send_wo, recv_wo, k_sem, v_sem):
        my = lax.axis_index("i")
        left = (my - 1) % N_DEV
        right = (my + 1) % N_DEV

        def kv_fetch(c, slot):
            return (
                pltpu.make_async_copy(k_hbm.at[c], k_buf.at[slot], k_sem.at[slot]),
                pltpu.make_async_copy(v_hbm.at[c], v_buf.at[slot], v_sem.at[slot]),
            )

        for d in kv_fetch(my, 0):
            d.start()

        barrier = pltpu.get_barrier_semaphore()
        for nbr in (left, right):
            pl.semaphore_signal(barrier, inc=1, device_id=(nbr,),
                                device_id_type=pl.DeviceIdType.MESH)
        pl.semaphore_wait(barrier, 2)

        def rdma_wq(src, c):
            return pltpu.make_async_remote_copy(
                src_ref=src, dst_ref=comm_wq.at[c],
                send_sem=send_wq.at[c], recv_sem=recv_wq.at[c],
                device_id=(right,), device_id_type=pl.DeviceIdType.MESH)

        def rdma_wo(src, c):
            return pltpu.make_async_remote_copy(
                src_ref=src, dst_ref=comm_wo.at[c],
                send_sem=send_wo.at[c], recv_sem=recv_wo.at[c],
                device_id=(right,), device_id_type=pl.DeviceIdType.MESH)

        rdma_wq(wq_ref, my).start()
        rdma_wo(wo_ref, my).start()

        rows = lax.broadcasted_iota(jnp.int32, (SQ, SKV), 0) // DH
        cols = lax.broadcasted_iota(jnp.int32, (SQ, SKV), 1) // DH
        mask = (rows % 4) == (cols % 4)
        mask = jnp.logical_or(mask, jnp.logical_and(rows == cols, my == 0))
        bias = jnp.where(mask, 0.0, NEG).astype(F32)

        acc = [None, None]
        for h in range(N_DEV):
            r = (my - h) % N_DEV
            slot = h % 2
            if h > 0:
                d_wq = rdma_wq(comm_wq.at[r], r)
                d_wo = rdma_wo(comm_wo.at[r], r)
                d_wq.wait_recv()
                d_wo.wait_recv()
                if h < N_DEV - 1:
                    d_wq.start()
                    d_wo.start()
                wq_c = comm_wq[r]
                wo_c = comm_wo[r]
            else:
                wq_c = wq_ref[...]
                wo_c = wo_ref[...]

            for d in kv_fetch(r, slot):
                d.wait()
            if h < N_DEV - 1:
                for d in kv_fetch((my - h - 1) % N_DEV, 1 - slot):
                    d.start()

            for b in range(B):
                q_b = lax.dot_general(
                    x_ref[b], wq_c, (((1,), (0,)), ((), ())),
                    preferred_element_type=F32).astype(BF)
                k_cb = k_buf[slot, b]
                v_cb = v_buf[slot, b]
                ctx_parts = []
                for hl in range(H_PER):
                    qh = q_b[:, hl * DH:(hl + 1) * DH]
                    kh = k_cb[:, hl * DH:(hl + 1) * DH]
                    vh = v_cb[:, hl * DH:(hl + 1) * DH]
                    s = lax.dot_general(
                        qh, kh, (((1,), (1,)), ((), ())),
                        preferred_element_type=F32)
                    e = jnp.exp(s * SCALE + bias)
                    denom = jnp.sum(e, axis=-1, keepdims=True)
                    ctx_raw = lax.dot_general(
                        e.astype(BF), vh, (((1,), (0,)), ((), ())),
                        preferred_element_type=F32)
                    ctx_parts.append((ctx_raw / denom).astype(BF))
                ctx_b = jnp.concatenate(ctx_parts, axis=1)
                partial = lax.dot_general(
                    ctx_b, wo_c, (((1,), (0,)), ((), ())),
                    preferred_element_type=F32)
                acc[b] = partial if h == 0 else acc[b] + partial

        for b in range(B):
            out_ref[b, :, :] = acc[b]

        for h in range(N_DEV - 1):
            c = (my - h) % N_DEV
            rdma_wq(comm_wq.at[c], c).wait_send()
            rdma_wo(comm_wo.at[c], c).wait_send()

    return pl.pallas_call(
        body,
        out_shape=jax.ShapeDtypeStruct((B, SQ, D_MODEL), F32),
        in_specs=[
            pl.BlockSpec(memory_space=pltpu.VMEM),
            pl.BlockSpec(memory_space=pltpu.VMEM),
            pl.BlockSpec(memory_space=pltpu.MemorySpace.HBM),
            pl.BlockSpec(memory_space=pltpu.MemorySpace.HBM),
            pl.BlockSpec(memory_space=pltpu.VMEM),
        ],
        out_specs=pl.BlockSpec(memory_space=pltpu.VMEM),
        scratch_shapes=[
            pltpu.VMEM((N_DEV, D_MODEL, D_CHUNK), BF),
            pltpu.VMEM((N_DEV, D_CHUNK, D_MODEL), BF),
            pltpu.VMEM((2, B, SKV, D_CHUNK), BF),
            pltpu.VMEM((2, B, SKV, D_CHUNK), BF),
            pltpu.SemaphoreType.DMA((N_DEV,)),
            pltpu.SemaphoreType.DMA((N_DEV,)),
            pltpu.SemaphoreType.DMA((N_DEV,)),
            pltpu.SemaphoreType.DMA((N_DEV,)),
            pltpu.SemaphoreType.DMA((2,)),
            pltpu.SemaphoreType.DMA((2,)),
        ],
        compiler_params=pltpu.CompilerParams(
            collective_id=0,
            vmem_limit_bytes=100 * 1024 * 1024,
        ),
    )(x_b16, Wq_b16, K_r, V_r, Wo_b16)


# device time: 130086 ns/iter; 2.3231x vs baseline; 1.3698x over previous
import jax
import jax.numpy as jnp
from jax import lax
from jax.experimental import pallas as pl
from jax.experimental.pallas import tpu as pltpu

N_DEV = 8
B = 2
SQ = 512
SKV = 512
D_MODEL = 768
H_PER = 8
DH = 64
D_CHUNK = H_PER * DH
SCALE = 0.125
NEG = -1e9
BF = jnp.bfloat16
F32 = jnp.float32


def kernel(x, Wq, K_ext, V_ext, Wo):
    x_b16 = x.astype(BF)
    Wq_b16 = Wq.astype(BF)
    Wo_b16 = Wo.astype(BF)
    K_r = K_ext.reshape(B, SKV, N_DEV, D_CHUNK).transpose(2, 0, 1, 3).astype(BF)
    V_r = V_ext.reshape(B, SKV, N_DEV, D_CHUNK).transpose(2, 0, 1, 3).astype(BF)

    OFFS = [0, -1, 1, -2, 2, -3, 3, 4]

    def body(x_ref, wq_ref, k_hbm, v_hbm, wo_ref, out_ref,
             comm_wq, comm_wo, k_buf, v_buf,
             send_wq_r, send_wq_l, recv_wq,
             send_wo_r, send_wo_l, recv_wo, k_sem, v_sem):
        my = lax.axis_index("i")
        left = (my - 1) % N_DEV
        right = (my + 1) % N_DEV

        def kv_fetch(c, slot):
            return (
                pltpu.make_async_copy(k_hbm.at[c], k_buf.at[slot], k_sem.at[slot]),
                pltpu.make_async_copy(v_hbm.at[c], v_buf.at[slot], v_sem.at[slot]),
            )

        for d in kv_fetch(my, 0):
            d.start()

        barrier = pltpu.get_barrier_semaphore()
        for nbr in (left, right):
            pl.semaphore_signal(barrier, inc=1, device_id=(nbr,),
                                device_id_type=pl.DeviceIdType.MESH)
        pl.semaphore_wait(barrier, 2)

        def rdma_wq(src, c, go_right):
            return pltpu.make_async_remote_copy(
                src_ref=src, dst_ref=comm_wq.at[c],
                send_sem=(send_wq_r if go_right else send_wq_l).at[c],
                recv_sem=recv_wq.at[c],
                device_id=(right if go_right else left,),
                device_id_type=pl.DeviceIdType.MESH)

        def rdma_wo(src, c, go_right):
            return pltpu.make_async_remote_copy(
                src_ref=src, dst_ref=comm_wo.at[c],
                send_sem=(send_wo_r if go_right else send_wo_l).at[c],
                recv_sem=recv_wo.at[c],
                device_id=(right if go_right else left,),
                device_id_type=pl.DeviceIdType.MESH)

        for go_right in (True, False):
            rdma_wq(wq_ref, my, go_right).start()
            rdma_wo(wo_ref, my, go_right).start()

        rows = lax.broadcasted_iota(jnp.int32, (SQ, SKV), 0) // DH
        cols = lax.broadcasted_iota(jnp.int32, (SQ, SKV), 1) // DH
        mask = (rows % 4) == (cols % 4)
        mask = jnp.logical_or(mask, jnp.logical_and(rows == cols, my == 0))
        bias = jnp.where(mask, 0.0, NEG).astype(F32)

        acc = [None, None]
        for step, off in enumerate(OFFS):
            r = (my + off) % N_DEV
            slot = step % 2
            if off != 0:
                go_right = off < 0
                d_wq = rdma_wq(comm_wq.at[r], r, go_right)
                d_wo = rdma_wo(comm_wo.at[r], r, go_right)
                d_wq.wait_recv()
                d_wo.wait_recv()
                if (-off < 3) if go_right else (off < 4):
                    d_wq.start()
                    d_wo.start()
                wq_c = comm_wq[r]
                wo_c = comm_wo[r]
            else:
                wq_c = wq_ref[...]
                wo_c = wo_ref[...]

            for d in kv_fetch(r, slot):
                d.wait()
            if step < N_DEV - 1:
                for d in kv_fetch((my + OFFS[step + 1]) % N_DEV, 1 - slot):
                    d.start()

            for b in range(B):
                q_b = lax.dot_general(
                    x_ref[b], wq_c, (((1,), (0,)), ((), ())),
                    preferred_element_type=F32).astype(BF)
                k_cb = k_buf[slot, b]
                v_cb = v_buf[slot, b]
                ctx_parts = []
                for hl in range(H_PER):
                    qh = q_b[:, hl * DH:(hl + 1) * DH]
                    kh = k_cb[:, hl * DH:(hl + 1) * DH]
                    vh = v_cb[:, hl * DH:(hl + 1) * DH]
                    s = lax.dot_general(
                        qh, kh, (((1,), (1,)), ((), ())),
                        preferred_element_type=F32)
                    e = jnp.exp(s * SCALE + bias)
                    denom = jnp.sum(e, axis=-1, keepdims=True)
                    ctx_raw = lax.dot_general(
                        e.astype(BF), vh, (((1,), (0,)), ((), ())),
                        preferred_element_type=F32)
                    ctx_parts.append((ctx_raw / denom).astype(BF))
                ctx_b = jnp.concatenate(ctx_parts, axis=1)
                partial = lax.dot_general(
                    ctx_b, wo_c, (((1,), (0,)), ((), ())),
                    preferred_element_type=F32)
                acc[b] = partial if step == 0 else acc[b] + partial

        for b in range(B):
            out_ref[b, :, :] = acc[b]

        for off in (0, -1, -2):
            c = (my + off) % N_DEV
            rdma_wq(comm_wq.at[c], c, True).wait_send()
            rdma_wo(comm_wo.at[c], c, True).wait_send()
        for off in (0, 1, 2, 3):
            c = (my + off) % N_DEV
            rdma_wq(comm_wq.at[c], c, False).wait_send()
            rdma_wo(comm_wo.at[c], c, False).wait_send()

    return pl.pallas_call(
        body,
        out_shape=jax.ShapeDtypeStruct((B, SQ, D_MODEL), F32),
        in_specs=[
            pl.BlockSpec(memory_space=pltpu.VMEM),
            pl.BlockSpec(memory_space=pltpu.VMEM),
            pl.BlockSpec(memory_space=pltpu.MemorySpace.HBM),
            pl.BlockSpec(memory_space=pltpu.MemorySpace.HBM),
            pl.BlockSpec(memory_space=pltpu.VMEM),
        ],
        out_specs=pl.BlockSpec(memory_space=pltpu.VMEM),
        scratch_shapes=[
            pltpu.VMEM((N_DEV, D_MODEL, D_CHUNK), BF),
            pltpu.VMEM((N_DEV, D_CHUNK, D_MODEL), BF),
            pltpu.VMEM((2, B, SKV, D_CHUNK), BF),
            pltpu.VMEM((2, B, SKV, D_CHUNK), BF),
            pltpu.SemaphoreType.DMA((N_DEV,)),
            pltpu.SemaphoreType.DMA((N_DEV,)),
            pltpu.SemaphoreType.DMA((N_DEV,)),
            pltpu.SemaphoreType.DMA((N_DEV,)),
            pltpu.SemaphoreType.DMA((N_DEV,)),
            pltpu.SemaphoreType.DMA((N_DEV,)),
            pltpu.SemaphoreType.DMA((2,)),
            pltpu.SemaphoreType.DMA((2,)),
        ],
        compiler_params=pltpu.CompilerParams(
            collective_id=0,
            vmem_limit_bytes=100 * 1024 * 1024,
        ),
    )(x_b16, Wq_b16, K_r, V_r, Wo_b16)
